# baseline (speedup 1.0000x reference)
import jax
import jax.numpy as jnp
from jax.experimental import pallas as pl

_B, _N, _DIN, _H, _OUT = 64, 512, 300, 64, 8
_STEPS = 2


def _gnn_kernel(inputs_ref, support_ref, mask_ref,
                W_enc_ref, b_enc_ref,
                Wz0_ref, Wz1_ref, bz_ref,
                Wr0_ref, Wr1_ref, br_ref,
                Wh0_ref, Wh1_ref, bh_ref,
                W_att_ref, b_att_ref,
                W_emb_ref, b_emb_ref,
                W_out_ref, b_out_ref,
                out_ref):
    f32 = jnp.float32
    m = mask_ref[0]
    x = jnp.tanh(
        jnp.dot(inputs_ref[0], W_enc_ref[...], preferred_element_type=f32)
        + b_enc_ref[...])
    x = x * m
    sup = support_ref[0]
    for _ in range(_STEPS):
        a = jnp.dot(sup, x, preferred_element_type=f32)
        z = jax.nn.sigmoid(
            jnp.dot(a, Wz0_ref[...], preferred_element_type=f32)
            + jnp.dot(x, Wz1_ref[...], preferred_element_type=f32)
            + bz_ref[...])
        r = jax.nn.sigmoid(
            jnp.dot(a, Wr0_ref[...], preferred_element_type=f32)
            + jnp.dot(x, Wr1_ref[...], preferred_element_type=f32)
            + br_ref[...])
        h = jnp.tanh(
            jnp.dot(a, Wh0_ref[...], preferred_element_type=f32)
            + jnp.dot(r * x, Wh1_ref[...], preferred_element_type=f32)
            + bh_ref[...])
        x = ((1.0 - z) * x + z * h) * m
    att = jax.nn.sigmoid(
        jnp.dot(x, W_att_ref[...], preferred_element_type=f32) + b_att_ref[...])
    emb = jnp.tanh(
        jnp.dot(x, W_emb_ref[...], preferred_element_type=f32) + b_emb_ref[...])
    g = m * att * emb
    n_nodes = jnp.sum(m, axis=0, keepdims=True)
    pooled = (jnp.sum(g, axis=0, keepdims=True) / n_nodes
              + jnp.max(g + (m - 1.0) * 1e9, axis=0, keepdims=True))
    out_ref[0] = (jnp.dot(pooled, W_out_ref[...], preferred_element_type=f32)
                  + b_out_ref[...])


def _full(shape):
    return pl.BlockSpec(shape, lambda b: (0,) * len(shape))


@jax.jit
def kernel(inputs, support, mask, W_enc, b_enc, Wz0, Wz1, bz, Wr0, Wr1, br,
           Wh0, Wh1, bh, W_att, b_att, W_emb, b_emb, W_out, b_out):
    B, N, DIN = inputs.shape
    H = W_enc.shape[1]
    OUT = W_out.shape[1]
    b_enc2 = b_enc.reshape(1, H)
    bz2 = bz.reshape(1, H)
    br2 = br.reshape(1, H)
    bh2 = bh.reshape(1, H)
    b_att2 = b_att.reshape(1, H)
    b_emb2 = b_emb.reshape(1, H)
    b_out2 = b_out.reshape(1, OUT)

    grid = (B,)
    batch3 = lambda shape: pl.BlockSpec(shape, lambda b: (b, 0, 0))
    in_specs = [
        batch3((1, N, DIN)),
        batch3((1, N, N)),
        batch3((1, N, 1)),
        _full((DIN, H)), _full((1, H)),
        _full((H, H)), _full((H, H)), _full((1, H)),
        _full((H, H)), _full((H, H)), _full((1, H)),
        _full((H, H)), _full((H, H)), _full((1, H)),
        _full((H, H)), _full((1, H)),
        _full((H, H)), _full((1, H)),
        _full((H, OUT)), _full((1, OUT)),
    ]
    out_spec = pl.BlockSpec((1, 1, OUT), lambda b: (b, 0, 0))

    out = pl.pallas_call(
        _gnn_kernel,
        grid=grid,
        in_specs=in_specs,
        out_specs=out_spec,
        out_shape=jax.ShapeDtypeStruct((B, 1, OUT), jnp.float32),
    )(inputs, support, mask, W_enc, b_enc2, Wz0, Wz1, bz2, Wr0, Wr1, br2,
      Wh0, Wh1, bh2, W_att, b_att2, W_emb, b_emb2, W_out, b_out2)
    return out.reshape(B, OUT)


# speedup vs baseline: 1.2290x; 1.2290x over previous
import jax
import jax.numpy as jnp
from jax.experimental import pallas as pl

_STEPS = 2
_GPB = 2


def _gnn_kernel(inputs_ref, support_ref, mask_ref,
                W_enc_ref, b_enc_ref,
                Wzr_ref, bzr_ref,
                Whh_ref, bh_ref,
                Wae_ref, bae_ref,
                W_out_ref, b_out_ref,
                out_ref):
    f32 = jnp.float32
    H = Whh_ref.shape[1]
    for g in range(_GPB):
        m = mask_ref[g]
        x = jnp.tanh(
            jnp.dot(inputs_ref[g], W_enc_ref[...], preferred_element_type=f32)
            + b_enc_ref[...])
        x = x * m
        sup = support_ref[g]
        for _ in range(_STEPS):
            a = jnp.dot(sup, x, preferred_element_type=f32)
            zr = jax.nn.sigmoid(
                jnp.dot(jnp.concatenate([a, x], axis=1), Wzr_ref[...],
                        preferred_element_type=f32)
                + bzr_ref[...])
            z = zr[:, :H]
            r = zr[:, H:]
            h = jnp.tanh(
                jnp.dot(jnp.concatenate([a, r * x], axis=1), Whh_ref[...],
                        preferred_element_type=f32)
                + bh_ref[...])
            x = ((1.0 - z) * x + z * h) * m
        ae = jnp.dot(x, Wae_ref[...], preferred_element_type=f32) + bae_ref[...]
        g_nodes = m * jax.nn.sigmoid(ae[:, :H]) * jnp.tanh(ae[:, H:])
        n_nodes = jnp.sum(m, axis=0, keepdims=True)
        pooled = (jnp.sum(g_nodes, axis=0, keepdims=True) / n_nodes
                  + jnp.max(g_nodes + (m - 1.0) * 1e9, axis=0, keepdims=True))
        out_ref[g] = (jnp.dot(pooled, W_out_ref[...],
                              preferred_element_type=f32) + b_out_ref[...])


def _full(shape):
    return pl.BlockSpec(shape, lambda b: (0,) * len(shape))


@jax.jit
def kernel(inputs, support, mask, W_enc, b_enc, Wz0, Wz1, bz, Wr0, Wr1, br,
           Wh0, Wh1, bh, W_att, b_att, W_emb, b_emb, W_out, b_out):
    B, N, DIN = inputs.shape
    H = W_enc.shape[1]
    OUT = W_out.shape[1]
    b_enc2 = b_enc.reshape(1, H)
    Wzr = jnp.concatenate(
        [jnp.concatenate([Wz0, Wr0], axis=1),
         jnp.concatenate([Wz1, Wr1], axis=1)], axis=0)
    bzr = jnp.concatenate([bz, br]).reshape(1, 2 * H)
    Whh = jnp.concatenate([Wh0, Wh1], axis=0)
    bh2 = bh.reshape(1, H)
    Wae = jnp.concatenate([W_att, W_emb], axis=1)
    bae = jnp.concatenate([b_att, b_emb]).reshape(1, 2 * H)
    b_out2 = b_out.reshape(1, OUT)

    grid = (B // _GPB,)
    batch3 = lambda shape: pl.BlockSpec(shape, lambda b: (b, 0, 0))
    in_specs = [
        batch3((_GPB, N, DIN)),
        batch3((_GPB, N, N)),
        batch3((_GPB, N, 1)),
        _full((DIN, H)), _full((1, H)),
        _full((2 * H, 2 * H)), _full((1, 2 * H)),
        _full((2 * H, H)), _full((1, H)),
        _full((H, 2 * H)), _full((1, 2 * H)),
        _full((H, OUT)), _full((1, OUT)),
    ]
    out_spec = pl.BlockSpec((_GPB, 1, OUT), lambda b: (b, 0, 0))

    out = pl.pallas_call(
        _gnn_kernel,
        grid=grid,
        in_specs=in_specs,
        out_specs=out_spec,
        out_shape=jax.ShapeDtypeStruct((B, 1, OUT), jnp.float32),
    )(inputs, support, mask, W_enc, b_enc2, Wzr, bzr, Whh, bh2, Wae, bae,
      W_out, b_out2)
    return out.reshape(B, OUT)
